# baseline (speedup 1.0000x reference)
import jax
import jax.numpy as jnp
from jax import lax
from jax.experimental import pallas as pl
from jax.experimental.pallas import tpu as pltpu
from jax.experimental.pallas import tpu_sc as plsc

NUM_FIELDS = 26
FIELD_SIZE = 100000
EMBED_DIM = 64
BATCH = 16384
TOTAL_ROWS = NUM_FIELDS * FIELD_SIZE

_info = plsc.get_sparse_core_info()
NC, NS, L = _info.num_cores, _info.num_subcores, _info.num_lanes
NW = NC * NS

ROWS_PER_W = BATCH // NW
BLK_ROWS = 32
N_BLK = ROWS_PER_W // BLK_ROWS
BLK_LOOK = BLK_ROWS * NUM_FIELDS
JROWS = 13
JCOLS = BLK_LOOK // JROWS


def _tec_body(cat_hbm, w_hbm, out_hbm, ib2, jidx, rows,
              isem0, isem1, gsem0, gsem1, osem0, osem1):
    wid = lax.axis_index("s") * NC + lax.axis_index("c")
    base_look = wid * ROWS_PER_W * NUM_FIELDS
    iota = lax.iota(jnp.int32, L)
    isem = (isem0, isem1)
    gsem = (gsem0, gsem1)
    osem = (osem0, osem1)

    def idx_start(i, h):
        pltpu.async_copy(
            cat_hbm.at[pl.ds(base_look + i * BLK_LOOK, BLK_LOOK)],
            ib2.at[h], isem[h])

    def idx_wait(i, h):
        pltpu.make_async_copy(
            cat_hbm.at[pl.ds(base_look + i * BLK_LOOK, BLK_LOOK)],
            ib2.at[h], isem[h]).wait()

    def compute_jidx(h):
        for r in range(JROWS):
            def lane(m, _, r=r):
                o = pl.multiple_of(m * L, L)
                pcol = lax.rem(r * JCOLS + o + iota, NUM_FIELDS)
                raw = ib2[h, pl.ds(r * JCOLS + o, L)]
                jidx[h, r, pl.ds(o, L)] = raw + pcol * FIELD_SIZE
                return 0
            lax.fori_loop(0, JCOLS // L, lane, 0)

    def gather_start(h):
        for r in range(JROWS):
            pltpu.async_copy(w_hbm.at[jidx.at[h, r]],
                             rows.at[h, pl.ds(r * JCOLS, JCOLS)], gsem[h])

    def gather_wait(h):
        for r in range(JROWS):
            pltpu.make_async_copy(w_hbm.at[jidx.at[h, r]],
                                  rows.at[h, pl.ds(r * JCOLS, JCOLS)],
                                  gsem[h]).wait()

    def out_start(i, h):
        pltpu.async_copy(rows.at[h],
                         out_hbm.at[pl.ds(base_look + i * BLK_LOOK, BLK_LOOK)],
                         osem[h])

    def out_wait(i, h):
        pltpu.make_async_copy(
            rows.at[h],
            out_hbm.at[pl.ds(base_look + i * BLK_LOOK, BLK_LOOK)],
            osem[h]).wait()

    idx_start(0, 0)
    idx_wait(0, 0)
    compute_jidx(0)
    gather_start(0)
    idx_start(1, 1)

    for i in range(N_BLK):
        h = i % 2
        h2 = 1 - h
        if i + 1 < N_BLK:
            idx_wait(i + 1, h2)
            compute_jidx(h2)
        gather_wait(h)
        if i >= 1:
            out_wait(i - 1, h2)
        if i + 1 < N_BLK:
            gather_start(h2)
        out_start(i, h)
        if i + 2 < N_BLK:
            idx_start(i + 2, h)
    out_wait(N_BLK - 1, (N_BLK - 1) % 2)


@jax.jit
def kernel(categorical_inputs, weights):
    mesh = plsc.VectorSubcoreMesh(core_axis_name="c", subcore_axis_name="s")
    out2 = pl.kernel(
        _tec_body,
        out_type=jax.ShapeDtypeStruct((BATCH * NUM_FIELDS, EMBED_DIM),
                                      jnp.float32),
        mesh=mesh,
        scratch_types=[
            pltpu.VMEM((2, BLK_LOOK), jnp.int32),
            pltpu.VMEM((2, JROWS, JCOLS), jnp.int32),
            pltpu.VMEM((2, BLK_LOOK, EMBED_DIM), jnp.float32),
            pltpu.SemaphoreType.DMA,
            pltpu.SemaphoreType.DMA,
            pltpu.SemaphoreType.DMA,
            pltpu.SemaphoreType.DMA,
            pltpu.SemaphoreType.DMA,
            pltpu.SemaphoreType.DMA,
        ],
        compiler_params=pltpu.CompilerParams(use_tc_tiling_on_sc=False,
                                             needs_layout_passes=False),
    )(categorical_inputs.reshape(-1), weights)
    return out2.reshape(BATCH, NUM_FIELDS, EMBED_DIM)
